# Initial kernel scaffold; baseline (speedup 1.0000x reference)
import jax
import jax.numpy as jnp
from jax.experimental import pallas as pl


def kernel(x, pe_table):
    raise NotImplementedError("write your pallas kernel here")


# speedup vs baseline: 1.4923x; 1.4923x over previous
import jax
import jax.numpy as jnp
from jax.experimental import pallas as pl

_BS = 512


def _add_pe_block(x_ref, pe_ref, o_ref):
    o_ref[...] = x_ref[...] + pe_ref[...]


def kernel(x, pe_table):
    B, S, H = x.shape
    grid = (S // _BS, B)
    return pl.pallas_call(
        _add_pe_block,
        grid=grid,
        in_specs=[
            pl.BlockSpec((1, _BS, H), lambda s, b: (b, s, 0)),
            pl.BlockSpec((_BS, H), lambda s, b: (s, 0)),
        ],
        out_specs=pl.BlockSpec((1, _BS, H), lambda s, b: (b, s, 0)),
        out_shape=jax.ShapeDtypeStruct((B, S, H), x.dtype),
    )(x, pe_table)


# speedup vs baseline: 1.6674x; 1.1173x over previous
import jax
import jax.numpy as jnp
from jax.experimental import pallas as pl

_BS = 1024


def _add_pe_block(x_ref, pe_ref, o_ref):
    o_ref[...] = x_ref[...] + pe_ref[...]


def kernel(x, pe_table):
    B, S, H = x.shape
    grid = (S // _BS, B)
    return pl.pallas_call(
        _add_pe_block,
        grid=grid,
        in_specs=[
            pl.BlockSpec((1, _BS, H), lambda s, b: (b, s, 0)),
            pl.BlockSpec((_BS, H), lambda s, b: (s, 0)),
        ],
        out_specs=pl.BlockSpec((1, _BS, H), lambda s, b: (b, s, 0)),
        out_shape=jax.ShapeDtypeStruct((B, S, H), x.dtype),
    )(x, pe_table)


# speedup vs baseline: 1.7386x; 1.0427x over previous
import jax
import jax.numpy as jnp
from jax.experimental import pallas as pl

_BS = 2048


def _add_pe_block(x_ref, pe_ref, o_ref):
    o_ref[...] = x_ref[...] + pe_ref[...]


def kernel(x, pe_table):
    B, S, H = x.shape
    grid = (S // _BS, B)
    return pl.pallas_call(
        _add_pe_block,
        grid=grid,
        in_specs=[
            pl.BlockSpec((1, _BS, H), lambda s, b: (b, s, 0)),
            pl.BlockSpec((_BS, H), lambda s, b: (s, 0)),
        ],
        out_specs=pl.BlockSpec((1, _BS, H), lambda s, b: (b, s, 0)),
        out_shape=jax.ShapeDtypeStruct((B, S, H), x.dtype),
    )(x, pe_table)
